# baseline (speedup 1.0000x reference)
import functools

import jax
import jax.numpy as jnp
from jax import lax
from jax.experimental import pallas as pl
from jax.experimental.pallas import tpu as pltpu
from jax.experimental.pallas import tpu_sc as plsc

N = 50000
F = 16
NPAD = 48
NROWS = N + NPAD
E = 1600000
EROWS = 12544
EPAD = EROWS * 128
NWORK = 32
RW = EROWS // NWORK
CH = 8
NCHUNK = RW // CH
SUBROWS = NROWS // 16
NEG = -1e30
BN = 2944
GRID_N = NROWS // BN
KS = (40000, 32000, 25600, 20480)


def _sc_edge_body(xa, srcm, dstm, em, out,
                  sbuf, dbuf, ebuf, xbuf, zbuf, acc, sem):
    c = lax.axis_index("c")
    s = lax.axis_index("s")
    wid = s * 2 + c
    bf16 = jnp.bfloat16

    def zrow(r, _):
        zbuf[pl.ds(r * 2, 2)] = jnp.zeros((2, F), bf16)
        return 0
    lax.fori_loop(0, SUBROWS // 2, zrow, 0)
    pltpu.sync_copy(zbuf, acc.at[pl.ds(s * SUBROWS, SUBROWS)])
    plsc.subcore_barrier()

    row0 = wid * RW

    def chunk(ci, _):
        r0 = row0 + ci * CH
        pltpu.sync_copy(srcm.at[pl.ds(r0, CH)], sbuf)
        pltpu.sync_copy(dstm.at[pl.ds(r0, CH)], dbuf)
        pltpu.sync_copy(em.at[pl.ds(r0 * 128, CH * 128)], ebuf)

        def row(j, _):
            pltpu.async_copy(xa.at[sbuf.at[j]], xbuf, sem).wait()
            eb = j * 128
            for t in range(64):
                e = 2 * t
                xp = xbuf[pl.ds(e, 2)]
                ep = ebuf[pl.ds(eb + e, 2)]
                xbuf[pl.ds(e, 2)] = jnp.maximum(xp + ep, 0.0)
            pltpu.sync_copy(xbuf, acc.at[dbuf.at[j]], add=True)
            return 0
        lax.fori_loop(0, CH, row, 0)
        return 0
    lax.fori_loop(0, NCHUNK, chunk, 0)

    plsc.subcore_barrier()

    off = s * SUBROWS
    pltpu.sync_copy(acc.at[pl.ds(off, SUBROWS)], zbuf)
    pltpu.sync_copy(zbuf, out.at[c, pl.ds(off, SUBROWS)])


def _sc_edge(xa, srcm, dstm, em):
    mesh = plsc.VectorSubcoreMesh(core_axis_name="c", subcore_axis_name="s")
    return pl.kernel(
        _sc_edge_body,
        out_type=jax.ShapeDtypeStruct((2, NROWS, F), jnp.bfloat16),
        mesh=mesh,
        compiler_params=pltpu.CompilerParams(use_tc_tiling_on_sc=False),
        scratch_types=[
            pltpu.VMEM((CH, 128), jnp.int32),
            pltpu.VMEM((CH, 128), jnp.int32),
            pltpu.VMEM((CH * 128, F), jnp.bfloat16),
            pltpu.VMEM((128, F), jnp.bfloat16),
            pltpu.VMEM((SUBROWS, F), jnp.bfloat16),
            pltpu.VMEM_SHARED((NROWS, F), jnp.bfloat16),
            pltpu.SemaphoreType.DMA,
        ],
    )(xa, srcm, dstm, em)


EB = 8192


def _tcE_body(a_ref, wv_ref, bv_ref, e_ref):
    e = a_ref[...] * wv_ref[0] + bv_ref[0]
    e_ref[...] = e.astype(jnp.bfloat16)[None]


def _tcE(attr_col, wv4, bv4):
    return pl.pallas_call(
        _tcE_body,
        grid=(4, EPAD // EB),
        in_specs=[
            pl.BlockSpec((EB, 1), lambda l, i: (i, 0)),
            pl.BlockSpec((1, 1, F), lambda l, i: (l, 0, 0)),
            pl.BlockSpec((1, 1, F), lambda l, i: (l, 0, 0)),
        ],
        out_specs=pl.BlockSpec((1, EB, F), lambda l, i: (l, i, 0)),
        out_shape=jax.ShapeDtypeStruct((4, EPAD, F), jnp.bfloat16),
    )(attr_col, wv4, bv4)


def _tcA_body(x_ref, agg_ref, mask_ref, wa_ref, ba_ref, g_ref, bb_ref,
              wbm_ref, b2_ref, p_ref, xpreT_ref, ms_ref):
    xb = (x_ref[...] + agg_ref[0].astype(jnp.float32)
          + agg_ref[1].astype(jnp.float32))
    h = jnp.dot(xb, wa_ref[...], preferred_element_type=jnp.float32)
    h = (h + ba_ref[...]) / jnp.sqrt(jnp.float32(1.0 + 1e-5)) * g_ref[...] + bb_ref[...]
    h = jnp.maximum(h, 0.0)
    h = jnp.dot(h, wbm_ref[...], preferred_element_type=jnp.float32) + b2_ref[...]
    xpre = jnp.maximum(h, 0.0)
    xpreT = xpre.T
    xpreT_ref[...] = xpreT
    p = p_ref[...]
    pn = p / jnp.sqrt(jnp.sum(p * p))
    score = jnp.tanh(jnp.sum(xpreT * pn, axis=0, keepdims=True))
    ms_ref[...] = jnp.where(mask_ref[...] > 0, score, -jnp.inf)


def _tcA(x_nm, agg2, mask, wa, ba, g, bb, wbm, b2, p):
    return pl.pallas_call(
        _tcA_body,
        grid=(GRID_N,),
        in_specs=[
            pl.BlockSpec((BN, F), lambda i: (i, 0)),
            pl.BlockSpec((2, BN, F), lambda i: (0, i, 0)),
            pl.BlockSpec((1, BN), lambda i: (0, i)),
            pl.BlockSpec((F, F), lambda i: (0, 0)),
            pl.BlockSpec((1, F), lambda i: (0, 0)),
            pl.BlockSpec((1, F), lambda i: (0, 0)),
            pl.BlockSpec((1, F), lambda i: (0, 0)),
            pl.BlockSpec((F, F), lambda i: (0, 0)),
            pl.BlockSpec((1, F), lambda i: (0, 0)),
            pl.BlockSpec((F, 1), lambda i: (0, 0)),
        ],
        out_specs=[
            pl.BlockSpec((F, BN), lambda i: (0, i)),
            pl.BlockSpec((1, BN), lambda i: (0, i)),
        ],
        out_shape=[
            jax.ShapeDtypeStruct((F, NROWS), jnp.float32),
            jax.ShapeDtypeStruct((1, NROWS), jnp.float32),
        ],
    )(x_nm, agg2, mask, wa, ba, g, bb, wbm, b2, p)


def _tcB_body(k, ms_ref, sel_ref, nm_ref):
    s = ms_ref[...]
    ub = lax.bitcast_convert_type(s, jnp.uint32)
    sign = ub >> jnp.uint32(31)
    u = jnp.where(sign == jnp.uint32(1), ~ub, ub | jnp.uint32(0x80000000))
    one = jnp.uint32(1)

    def tstep(t, T):
        cand = T | (one << (jnp.uint32(31) - t.astype(jnp.uint32)))
        cnt = jnp.sum((u >= cand).astype(jnp.int32))
        return jnp.where(cnt >= k, cand, T)
    T = lax.fori_loop(0, 32, tstep, jnp.uint32(0))

    cnt_gt = jnp.sum((u > T).astype(jnp.int32))
    need = k - cnt_gt
    eq = (u == T)
    idx = lax.broadcasted_iota(jnp.int32, (1, NROWS), 1)

    def pstep(t, p):
        trial = p + (jnp.int32(1) << (jnp.int32(16) - t))
        g = jnp.sum((eq & (idx < trial)).astype(jnp.int32))
        return jnp.where(g < need, trial, p)
    p = lax.fori_loop(0, 17, pstep, jnp.int32(0))

    keep = (u > T) | (eq & (idx <= p))
    nm_ref[...] = keep.astype(jnp.float32)
    sel_ref[...] = jnp.where(keep, s, 0.0)


def _tcB(ms, k):
    return pl.pallas_call(
        functools.partial(_tcB_body, k),
        out_shape=[
            jax.ShapeDtypeStruct((1, NROWS), jnp.float32),
            jax.ShapeDtypeStruct((1, NROWS), jnp.float32),
        ],
    )(ms)


def _tcC_body(xpreT_ref, sel_ref, nm_ref, xa_ref, xab_ref, rmax_ref, rsum_ref):
    i = pl.program_id(0)
    xoutT = xpreT_ref[...] * sel_ref[...]
    nm = nm_ref[...]
    xaug = jnp.where(nm > 0, xoutT, NEG).T
    xa_ref[...] = xaug
    xab_ref[...] = xaug.astype(jnp.bfloat16)
    bmax = jnp.max(jnp.where(nm > 0, xoutT, -jnp.inf), axis=1, keepdims=True)
    bsum = jnp.sum(xoutT, axis=1, keepdims=True)

    @pl.when(i == 0)
    def _():
        rmax_ref[...] = bmax
        rsum_ref[...] = bsum

    @pl.when(i > 0)
    def _():
        rmax_ref[...] = jnp.maximum(rmax_ref[...], bmax)
        rsum_ref[...] = rsum_ref[...] + bsum


def _tcC(xpreT, sel, nm):
    return pl.pallas_call(
        _tcC_body,
        grid=(GRID_N,),
        in_specs=[
            pl.BlockSpec((F, BN), lambda i: (0, i)),
            pl.BlockSpec((1, BN), lambda i: (0, i)),
            pl.BlockSpec((1, BN), lambda i: (0, i)),
        ],
        out_specs=[
            pl.BlockSpec((BN, F), lambda i: (i, 0)),
            pl.BlockSpec((BN, F), lambda i: (i, 0)),
            pl.BlockSpec((F, 1), lambda i: (0, 0)),
            pl.BlockSpec((F, 1), lambda i: (0, 0)),
        ],
        out_shape=[
            jax.ShapeDtypeStruct((NROWS, F), jnp.float32),
            jax.ShapeDtypeStruct((NROWS, F), jnp.bfloat16),
            jax.ShapeDtypeStruct((F, 1), jnp.float32),
            jax.ShapeDtypeStruct((F, 1), jnp.float32),
        ],
    )(xpreT, sel, nm)


def _tcD_body(rm0, rs0, rm1, rs1, rm2, rs2, rm3, rs3,
              w1_ref, b1_ref, w2_ref, b2_ref, w3_ref, b3_ref, out_ref):
    reads = None
    for rm, rs, k in ((rm0, rs0, KS[0]), (rm1, rs1, KS[1]),
                      (rm2, rs2, KS[2]), (rm3, rs3, KS[3])):
        r = jnp.concatenate([rm[...].T, rs[...].T / jnp.float32(k)], axis=1)
        reads = r if reads is None else reads + r
    h = jnp.maximum(jnp.dot(reads, w1_ref[...],
                            preferred_element_type=jnp.float32) + b1_ref[...], 0.0)
    h = jnp.maximum(jnp.dot(h, w2_ref[...],
                            preferred_element_type=jnp.float32) + b2_ref[...], 0.0)
    z = jnp.dot(h, w3_ref[...], preferred_element_type=jnp.float32) + b3_ref[...]
    zm = jnp.max(z, axis=1, keepdims=True)
    zs = z - zm
    out_ref[...] = zs - jnp.log(jnp.sum(jnp.exp(zs), axis=1, keepdims=True))


def _tcD(rstats, w1, b1, w2, b2, w3, b3):
    args = []
    for rm, rs in rstats:
        args += [rm, rs]
    return pl.pallas_call(
        _tcD_body,
        out_shape=jax.ShapeDtypeStruct((1, 5), jnp.float32),
    )(*args, w1, b1, w2, b2, w3, b3)


def kernel(x, edge_index, edge_attr, batch, params):
    del batch
    f32 = jnp.float32
    src = edge_index[0]
    dst = edge_index[1]
    padi = (N + (jnp.arange(EPAD - E, dtype=jnp.int32) % NPAD)).astype(jnp.int32)
    srcm = jnp.concatenate([src, padi]).reshape(EROWS, 128)
    dstm = jnp.concatenate([dst, padi]).reshape(EROWS, 128)
    attr_col = jnp.concatenate([edge_attr[:, 0],
                                jnp.zeros((EPAD - E,), f32)])[:, None]

    xcol = jnp.concatenate([x[:, 0], jnp.full((NPAD,), NEG, f32)])[:, None]
    rest = jnp.concatenate([jnp.zeros((N, F - 1), f32),
                            jnp.full((NPAD, F - 1), NEG, f32)], axis=0)
    x_nm = jnp.concatenate([xcol, rest], axis=1)
    mask = jnp.concatenate([jnp.ones((N,), f32),
                            jnp.zeros((NPAD,), f32)])[None, :]
    x_bf = x_nm.astype(jnp.bfloat16)

    wvs, bvs = [], []
    for i in range(4):
        we = params['We%d' % i][0]
        be = params['be%d' % i]
        d = we.shape[0]
        wvs.append(jnp.concatenate([we, jnp.zeros((F - d,), f32)]) if d < F else we)
        bvs.append(jnp.concatenate([be, jnp.zeros((F - d,), f32)]) if d < F else be)
    embeds = _tcE(attr_col, jnp.stack(wvs)[:, None], jnp.stack(bvs)[:, None])

    rstats = []
    for i, k in enumerate(KS):
        wa = params['Wa%d' % i]
        if wa.shape[0] < F:
            wa = jnp.concatenate([wa, jnp.zeros((F - wa.shape[0], F), f32)], axis=0)

        agg2 = _sc_edge(x_bf, srcm, dstm, embeds[i])
        xpreT, ms = _tcA(x_nm, agg2, mask,
                         wa, params['ba%d' % i][None, :], params['g%d' % i][None, :],
                         params['bb%d' % i][None, :], params['Wb%d' % i],
                         params['b2_%d' % i][None, :], params['p%d' % i][:, None])
        sel, nm = _tcB(ms, k)
        x_nm, x_bf, rmax, rsum = _tcC(xpreT, sel, nm)
        mask = nm
        rstats.append((rmax, rsum))

    return _tcD(rstats, params['Wl1'], params['bl1'][None, :],
                params['Wl2'], params['bl2'][None, :],
                params['Wl3'], params['bl3'][None, :])


# speedup vs baseline: 4.7705x; 4.7705x over previous
import functools

import jax
import jax.numpy as jnp
from jax import lax
from jax.experimental import pallas as pl
from jax.experimental.pallas import tpu as pltpu
from jax.experimental.pallas import tpu_sc as plsc

N = 50000
F = 16
NPAD = 48
NROWS = N + NPAD
E = 1600000
EROWS = 12544
EPAD = EROWS * 128
NWORK = 32
RW = EROWS // NWORK
CH = 56
NCHUNK = RW // CH
PAIRS = CH // 4
SUBROWS = NROWS // 16
NEG = -1e30
BN = 2944
GRID_N = NROWS // BN
KS = (40000, 32000, 25600, 20480)


def _sc_edge_body(xa, srcm, dstm, attrm, wb, out,
                  sbuf, dbuf, abuf, xb0, xb1, xb2, xb3, wbbuf, zbuf, acc,
                  gs0, gs1, gs2, gs3, ss0, ss1, ss2, ss3):
    c = lax.axis_index("c")
    s = lax.axis_index("s")
    wid = s * 2 + c
    xbufs = (xb0, xb1, xb2, xb3)
    gsems = (gs0, gs1, gs2, gs3)
    ssems = (ss0, ss1, ss2, ss3)

    def zrow(r, _):
        zbuf[r] = jnp.zeros((F,), jnp.float32)
        return 0
    lax.fori_loop(0, SUBROWS, zrow, 0)
    pltpu.sync_copy(zbuf, acc.at[pl.ds(s * SUBROWS, SUBROWS)])

    pltpu.sync_copy(wb, wbbuf)
    plsc.subcore_barrier()

    wv = wbbuf[0]
    bv = wbbuf[1]
    row0 = wid * RW

    def chunk(ci, _):
        r0 = row0 + ci * CH
        pltpu.sync_copy(srcm.at[pl.ds(r0, CH)], sbuf)
        pltpu.sync_copy(dstm.at[pl.ds(r0, CH)], dbuf)
        pltpu.sync_copy(attrm.at[pl.ds(r0, CH)], abuf)

        for u in range(4):
            pltpu.async_copy(xa.at[sbuf.at[u]], xbufs[u], gsems[u])

        def body(t, _):
            for u in range(4):
                j = t * 4 + u
                pltpu.make_async_copy(xa.at[sbuf.at[j]], xbufs[u],
                                      gsems[u]).wait()
                xbuf = xbufs[u]
                for e0 in range(0, 128, 16):
                    avec = abuf[j, pl.ds(e0, 16)]
                    for tt in range(16):
                        e = e0 + tt
                        xbuf[e] = jnp.maximum(
                            xbuf[e] + (avec[tt] * wv + bv), 0.0)
                pltpu.async_copy(xbuf, acc.at[dbuf.at[j]], ssems[u],
                                 add=True)

            @pl.when(t < PAIRS - 1)
            def _():
                for u in range(4):
                    j = t * 4 + u
                    pltpu.make_async_copy(xbufs[u], acc.at[dbuf.at[j]],
                                          ssems[u]).wait()
                    pltpu.async_copy(xa.at[sbuf.at[j + 4]], xbufs[u],
                                     gsems[u])
            return 0
        lax.fori_loop(0, PAIRS, body, 0)

        for u in range(4):
            j = (PAIRS - 1) * 4 + u
            pltpu.make_async_copy(xbufs[u], acc.at[dbuf.at[j]],
                                  ssems[u]).wait()
        return 0
    lax.fori_loop(0, NCHUNK, chunk, 0)

    plsc.subcore_barrier()

    off = s * SUBROWS
    pltpu.sync_copy(acc.at[pl.ds(off, SUBROWS)], zbuf)
    pltpu.sync_copy(zbuf, out.at[c, pl.ds(off, SUBROWS)])


def _sc_edge(xa, srcm, dstm, attrm, wb):
    mesh = plsc.VectorSubcoreMesh(core_axis_name="c", subcore_axis_name="s")
    return pl.kernel(
        _sc_edge_body,
        out_type=jax.ShapeDtypeStruct((2, NROWS, F), jnp.float32),
        mesh=mesh,
        compiler_params=pltpu.CompilerParams(use_tc_tiling_on_sc=False),
        scratch_types=[
            pltpu.VMEM((CH, 128), jnp.int32),
            pltpu.VMEM((CH, 128), jnp.int32),
            pltpu.VMEM((CH, 128), jnp.float32),
            pltpu.VMEM((128, F), jnp.float32),
            pltpu.VMEM((128, F), jnp.float32),
            pltpu.VMEM((128, F), jnp.float32),
            pltpu.VMEM((128, F), jnp.float32),
            pltpu.VMEM((2, F), jnp.float32),
            pltpu.VMEM((SUBROWS, F), jnp.float32),
            pltpu.VMEM_SHARED((NROWS, F), jnp.float32),
        ] + [pltpu.SemaphoreType.DMA] * 8,
    )(xa, srcm, dstm, attrm, wb)


def _tcA_body(x_ref, agg_ref, mask_ref, wa_ref, ba_ref, g_ref, bb_ref,
              wbm_ref, b2_ref, p_ref, xpreT_ref, ms_ref):
    xb = x_ref[...] + agg_ref[0] + agg_ref[1]
    h = jnp.dot(xb, wa_ref[...], preferred_element_type=jnp.float32)
    h = (h + ba_ref[...]) / jnp.sqrt(jnp.float32(1.0 + 1e-5)) * g_ref[...] + bb_ref[...]
    h = jnp.maximum(h, 0.0)
    h = jnp.dot(h, wbm_ref[...], preferred_element_type=jnp.float32) + b2_ref[...]
    xpre = jnp.maximum(h, 0.0)
    xpreT = xpre.T
    xpreT_ref[...] = xpreT
    p = p_ref[...]
    pn = p / jnp.sqrt(jnp.sum(p * p))
    score = jnp.tanh(jnp.sum(xpreT * pn, axis=0, keepdims=True))
    ms_ref[...] = jnp.where(mask_ref[...] > 0, score, -jnp.inf)


def _tcA(x_nm, agg2, mask, wa, ba, g, bb, wbm, b2, p):
    return pl.pallas_call(
        _tcA_body,
        grid=(GRID_N,),
        in_specs=[
            pl.BlockSpec((BN, F), lambda i: (i, 0)),
            pl.BlockSpec((2, BN, F), lambda i: (0, i, 0)),
            pl.BlockSpec((1, BN), lambda i: (0, i)),
            pl.BlockSpec((F, F), lambda i: (0, 0)),
            pl.BlockSpec((1, F), lambda i: (0, 0)),
            pl.BlockSpec((1, F), lambda i: (0, 0)),
            pl.BlockSpec((1, F), lambda i: (0, 0)),
            pl.BlockSpec((F, F), lambda i: (0, 0)),
            pl.BlockSpec((1, F), lambda i: (0, 0)),
            pl.BlockSpec((F, 1), lambda i: (0, 0)),
        ],
        out_specs=[
            pl.BlockSpec((F, BN), lambda i: (0, i)),
            pl.BlockSpec((1, BN), lambda i: (0, i)),
        ],
        out_shape=[
            jax.ShapeDtypeStruct((F, NROWS), jnp.float32),
            jax.ShapeDtypeStruct((1, NROWS), jnp.float32),
        ],
    )(x_nm, agg2, mask, wa, ba, g, bb, wbm, b2, p)


def _tcB_body(k, ms_ref, sel_ref, nm_ref):
    s = ms_ref[...]
    ub = lax.bitcast_convert_type(s, jnp.uint32)
    sign = ub >> jnp.uint32(31)
    u = jnp.where(sign == jnp.uint32(1), ~ub, ub | jnp.uint32(0x80000000))
    one = jnp.uint32(1)

    def tstep(t, T):
        cand = T | (one << (jnp.uint32(31) - t.astype(jnp.uint32)))
        cnt = jnp.sum((u >= cand).astype(jnp.int32))
        return jnp.where(cnt >= k, cand, T)
    T = lax.fori_loop(0, 32, tstep, jnp.uint32(0))

    cnt_gt = jnp.sum((u > T).astype(jnp.int32))
    need = k - cnt_gt
    eq = (u == T)
    idx = lax.broadcasted_iota(jnp.int32, (1, NROWS), 1)

    def pstep(t, p):
        trial = p + (jnp.int32(1) << (jnp.int32(16) - t))
        g = jnp.sum((eq & (idx < trial)).astype(jnp.int32))
        return jnp.where(g < need, trial, p)
    p = lax.fori_loop(0, 17, pstep, jnp.int32(0))

    keep = (u > T) | (eq & (idx <= p))
    nm_ref[...] = keep.astype(jnp.float32)
    sel_ref[...] = jnp.where(keep, s, 0.0)


def _tcB(ms, k):
    return pl.pallas_call(
        functools.partial(_tcB_body, k),
        out_shape=[
            jax.ShapeDtypeStruct((1, NROWS), jnp.float32),
            jax.ShapeDtypeStruct((1, NROWS), jnp.float32),
        ],
    )(ms)


def _tcC_body(xpreT_ref, sel_ref, nm_ref, xa_ref, rmax_ref, rsum_ref):
    i = pl.program_id(0)
    xoutT = xpreT_ref[...] * sel_ref[...]
    nm = nm_ref[...]
    xa_ref[...] = jnp.where(nm > 0, xoutT, NEG).T
    bmax = jnp.max(jnp.where(nm > 0, xoutT, -jnp.inf), axis=1, keepdims=True)
    bsum = jnp.sum(xoutT, axis=1, keepdims=True)

    @pl.when(i == 0)
    def _():
        rmax_ref[...] = bmax
        rsum_ref[...] = bsum

    @pl.when(i > 0)
    def _():
        rmax_ref[...] = jnp.maximum(rmax_ref[...], bmax)
        rsum_ref[...] = rsum_ref[...] + bsum


def _tcC(xpreT, sel, nm):
    return pl.pallas_call(
        _tcC_body,
        grid=(GRID_N,),
        in_specs=[
            pl.BlockSpec((F, BN), lambda i: (0, i)),
            pl.BlockSpec((1, BN), lambda i: (0, i)),
            pl.BlockSpec((1, BN), lambda i: (0, i)),
        ],
        out_specs=[
            pl.BlockSpec((BN, F), lambda i: (i, 0)),
            pl.BlockSpec((F, 1), lambda i: (0, 0)),
            pl.BlockSpec((F, 1), lambda i: (0, 0)),
        ],
        out_shape=[
            jax.ShapeDtypeStruct((NROWS, F), jnp.float32),
            jax.ShapeDtypeStruct((F, 1), jnp.float32),
            jax.ShapeDtypeStruct((F, 1), jnp.float32),
        ],
    )(xpreT, sel, nm)


def _tcD_body(rm0, rs0, rm1, rs1, rm2, rs2, rm3, rs3,
              w1_ref, b1_ref, w2_ref, b2_ref, w3_ref, b3_ref, out_ref):
    reads = None
    for rm, rs, k in ((rm0, rs0, KS[0]), (rm1, rs1, KS[1]),
                      (rm2, rs2, KS[2]), (rm3, rs3, KS[3])):
        r = jnp.concatenate([rm[...].T, rs[...].T / jnp.float32(k)], axis=1)
        reads = r if reads is None else reads + r
    h = jnp.maximum(jnp.dot(reads, w1_ref[...],
                            preferred_element_type=jnp.float32) + b1_ref[...], 0.0)
    h = jnp.maximum(jnp.dot(h, w2_ref[...],
                            preferred_element_type=jnp.float32) + b2_ref[...], 0.0)
    z = jnp.dot(h, w3_ref[...], preferred_element_type=jnp.float32) + b3_ref[...]
    zm = jnp.max(z, axis=1, keepdims=True)
    zs = z - zm
    out_ref[...] = zs - jnp.log(jnp.sum(jnp.exp(zs), axis=1, keepdims=True))


def _tcD(rstats, w1, b1, w2, b2, w3, b3):
    args = []
    for rm, rs in rstats:
        args += [rm, rs]
    return pl.pallas_call(
        _tcD_body,
        out_shape=jax.ShapeDtypeStruct((1, 5), jnp.float32),
    )(*args, w1, b1, w2, b2, w3, b3)


def kernel(x, edge_index, edge_attr, batch, params):
    del batch
    f32 = jnp.float32
    src = edge_index[0]
    dst = edge_index[1]
    padi = (N + (jnp.arange(EPAD - E, dtype=jnp.int32) % NPAD)).astype(jnp.int32)
    srcm = jnp.concatenate([src, padi]).reshape(EROWS, 128)
    dstm = jnp.concatenate([dst, padi]).reshape(EROWS, 128)
    attrm = jnp.concatenate([edge_attr[:, 0],
                             jnp.zeros((EPAD - E,), f32)]).reshape(EROWS, 128)

    xcol = jnp.concatenate([x[:, 0], jnp.full((NPAD,), NEG, f32)])[:, None]
    rest = jnp.concatenate([jnp.zeros((N, F - 1), f32),
                            jnp.full((NPAD, F - 1), NEG, f32)], axis=0)
    x_nm = jnp.concatenate([xcol, rest], axis=1)
    mask = jnp.concatenate([jnp.ones((N,), f32),
                            jnp.zeros((NPAD,), f32)])[None, :]

    rstats = []
    for i, k in enumerate(KS):
        we = params['We%d' % i][0]
        be = params['be%d' % i]
        d = we.shape[0]
        wv = jnp.concatenate([we, jnp.zeros((F - d,), f32)]) if d < F else we
        bv = jnp.concatenate([be, jnp.zeros((F - d,), f32)]) if d < F else be
        wb = jnp.stack([wv, bv])
        wa = params['Wa%d' % i]
        if wa.shape[0] < F:
            wa = jnp.concatenate([wa, jnp.zeros((F - wa.shape[0], F), f32)], axis=0)

        agg2 = _sc_edge(x_nm, srcm, dstm, attrm, wb)
        xpreT, ms = _tcA(x_nm, agg2, mask,
                         wa, params['ba%d' % i][None, :], params['g%d' % i][None, :],
                         params['bb%d' % i][None, :], params['Wb%d' % i],
                         params['b2_%d' % i][None, :], params['p%d' % i][:, None])
        sel, nm = _tcB(ms, k)
        x_nm, rmax, rsum = _tcC(xpreT, sel, nm)
        mask = nm
        rstats.append((rmax, rsum))

    return _tcD(rstats, params['Wl1'], params['bl1'][None, :],
                params['Wl2'], params['bl2'][None, :],
                params['Wl3'], params['bl3'][None, :])


# speedup vs baseline: 4.8107x; 1.0084x over previous
import functools

import jax
import jax.numpy as jnp
from jax import lax
from jax.experimental import pallas as pl
from jax.experimental.pallas import tpu as pltpu
from jax.experimental.pallas import tpu_sc as plsc

N = 50000
F = 16
NPAD = 48
NROWS = N + NPAD
E = 1600000
EROWS = 12544
EPAD = EROWS * 128
NWORK = 32
RW = EROWS // NWORK
CH = 56
NCHUNK = RW // CH
PAIRS = CH // 4
SUBROWS = NROWS // 16
NEG = -1e30
BN = 2944
GRID_N = NROWS // BN
KS = (40000, 32000, 25600, 20480)


def _sc_edge_body(xa, srcm, dstm, attrm, wb, out,
                  sbuf, dbuf, abuf, xb0, xb1, xb2, xb3, wbbuf, zbuf, acc,
                  gs0, gs1, gs2, gs3, ss0, ss1, ss2, ss3):
    c = lax.axis_index("c")
    s = lax.axis_index("s")
    wid = s * 2 + c
    xbufs = (xb0, xb1, xb2, xb3)
    gsems = (gs0, gs1, gs2, gs3)
    ssems = (ss0, ss1, ss2, ss3)

    def zrow(r, _):
        zbuf[r] = jnp.zeros((F,), jnp.float32)
        return 0
    lax.fori_loop(0, SUBROWS, zrow, 0)
    pltpu.sync_copy(zbuf, acc.at[pl.ds(s * SUBROWS, SUBROWS)])

    pltpu.sync_copy(wb, wbbuf)
    plsc.subcore_barrier()

    wv = wbbuf[0]
    bv = wbbuf[1]
    row0 = wid * RW

    def chunk(ci, _):
        r0 = row0 + ci * CH
        pltpu.sync_copy(srcm.at[pl.ds(r0, CH)], sbuf)
        pltpu.sync_copy(dstm.at[pl.ds(r0, CH)], dbuf)
        pltpu.sync_copy(attrm.at[pl.ds(r0, CH)], abuf)

        for u in range(4):
            pltpu.async_copy(xa.at[sbuf.at[u]], xbufs[u], gsems[u])

        def body(t, _):
            for u in range(4):
                j = t * 4 + u
                pltpu.make_async_copy(xa.at[sbuf.at[j]], xbufs[u],
                                      gsems[u]).wait()
                xbuf = xbufs[u]
                for e0 in range(0, 128, 16):
                    avec = abuf[j, pl.ds(e0, 16)]
                    for tt in range(16):
                        e = e0 + tt
                        xbuf[e] = jnp.maximum(
                            xbuf[e] + (avec[tt] * wv + bv), 0.0)
                pltpu.async_copy(xbuf, acc.at[dbuf.at[j]], ssems[u],
                                 add=True)

            @pl.when(t < PAIRS - 1)
            def _():
                for u in range(4):
                    j = t * 4 + u
                    pltpu.make_async_copy(xbufs[u], acc.at[dbuf.at[j]],
                                          ssems[u]).wait()
                    pltpu.async_copy(xa.at[sbuf.at[j + 4]], xbufs[u],
                                     gsems[u])
            return 0
        lax.fori_loop(0, PAIRS, body, 0)

        for u in range(4):
            j = (PAIRS - 1) * 4 + u
            pltpu.make_async_copy(xbufs[u], acc.at[dbuf.at[j]],
                                  ssems[u]).wait()
        return 0
    lax.fori_loop(0, NCHUNK, chunk, 0)

    plsc.subcore_barrier()

    off = s * SUBROWS
    pltpu.sync_copy(acc.at[pl.ds(off, SUBROWS)], zbuf)
    pltpu.sync_copy(zbuf, out.at[c, pl.ds(off, SUBROWS)])


def _sc_edge(xa, srcm, dstm, attrm, wb):
    mesh = plsc.VectorSubcoreMesh(core_axis_name="c", subcore_axis_name="s")
    return pl.kernel(
        _sc_edge_body,
        out_type=jax.ShapeDtypeStruct((2, NROWS, F), jnp.float32),
        mesh=mesh,
        compiler_params=pltpu.CompilerParams(use_tc_tiling_on_sc=False),
        scratch_types=[
            pltpu.VMEM((CH, 128), jnp.int32),
            pltpu.VMEM((CH, 128), jnp.int32),
            pltpu.VMEM((CH, 128), jnp.float32),
            pltpu.VMEM((128, F), jnp.float32),
            pltpu.VMEM((128, F), jnp.float32),
            pltpu.VMEM((128, F), jnp.float32),
            pltpu.VMEM((128, F), jnp.float32),
            pltpu.VMEM((2, F), jnp.float32),
            pltpu.VMEM((SUBROWS, F), jnp.float32),
            pltpu.VMEM_SHARED((NROWS, F), jnp.float32),
        ] + [pltpu.SemaphoreType.DMA] * 8,
    )(xa, srcm, dstm, attrm, wb)


def _tcA_body(x_ref, agg_ref, mask_ref, wa_ref, ba_ref, g_ref, bb_ref,
              wbm_ref, b2_ref, p_ref, xpreT_ref, ms_ref):
    xb = x_ref[...] + agg_ref[0] + agg_ref[1]
    h = jnp.dot(xb, wa_ref[...], preferred_element_type=jnp.float32)
    h = (h + ba_ref[...]) / jnp.sqrt(jnp.float32(1.0 + 1e-5)) * g_ref[...] + bb_ref[...]
    h = jnp.maximum(h, 0.0)
    h = jnp.dot(h, wbm_ref[...], preferred_element_type=jnp.float32) + b2_ref[...]
    xpre = jnp.maximum(h, 0.0)
    xpreT = xpre.T
    xpreT_ref[...] = xpreT
    p = p_ref[...]
    pn = p / jnp.sqrt(jnp.sum(p * p))
    score = jnp.tanh(jnp.sum(xpreT * pn, axis=0, keepdims=True))
    ms_ref[...] = jnp.where(mask_ref[...] > 0, score, -jnp.inf)


def _tcA(x_nm, agg2, mask, wa, ba, g, bb, wbm, b2, p):
    return pl.pallas_call(
        _tcA_body,
        grid=(GRID_N,),
        in_specs=[
            pl.BlockSpec((BN, F), lambda i: (i, 0)),
            pl.BlockSpec((2, BN, F), lambda i: (0, i, 0)),
            pl.BlockSpec((1, BN), lambda i: (0, i)),
            pl.BlockSpec((F, F), lambda i: (0, 0)),
            pl.BlockSpec((1, F), lambda i: (0, 0)),
            pl.BlockSpec((1, F), lambda i: (0, 0)),
            pl.BlockSpec((1, F), lambda i: (0, 0)),
            pl.BlockSpec((F, F), lambda i: (0, 0)),
            pl.BlockSpec((1, F), lambda i: (0, 0)),
            pl.BlockSpec((F, 1), lambda i: (0, 0)),
        ],
        out_specs=[
            pl.BlockSpec((F, BN), lambda i: (0, i)),
            pl.BlockSpec((1, BN), lambda i: (0, i)),
        ],
        out_shape=[
            jax.ShapeDtypeStruct((F, NROWS), jnp.float32),
            jax.ShapeDtypeStruct((1, NROWS), jnp.float32),
        ],
    )(x_nm, agg2, mask, wa, ba, g, bb, wbm, b2, p)


def _tcB_body(k, ms_ref, sel_ref, nm_ref):
    s = ms_ref[...]
    ub = lax.bitcast_convert_type(s, jnp.uint32)
    sign = ub >> jnp.uint32(31)
    u = jnp.where(sign == jnp.uint32(1), ~ub, ub | jnp.uint32(0x80000000))
    one = jnp.uint32(1)

    def tstep(t, T):
        hi = jnp.uint32(31) - 2 * t.astype(jnp.uint32)
        c1 = T | (one << hi)
        c2 = T | (one << (hi - 1))
        c3 = c1 | (one << (hi - 1))
        n1 = jnp.sum((u >= c1).astype(jnp.int32))
        n2 = jnp.sum((u >= c2).astype(jnp.int32))
        n3 = jnp.sum((u >= c3).astype(jnp.int32))
        return jnp.where(n3 >= k, c3,
                         jnp.where(n1 >= k, c1,
                                   jnp.where(n2 >= k, c2, T)))
    T = lax.fori_loop(0, 16, tstep, jnp.uint32(0))

    cnt_gt = jnp.sum((u > T).astype(jnp.int32))
    total_eq = jnp.sum((u == T).astype(jnp.int32))
    need = k - cnt_gt
    eq = (u == T)
    idx = lax.broadcasted_iota(jnp.int32, (1, NROWS), 1)

    def tie_search():
        def pstep(t, p):
            trial = p + (jnp.int32(1) << (jnp.int32(16) - t))
            g = jnp.sum((eq & (idx < trial)).astype(jnp.int32))
            return jnp.where(g < need, trial, p)
        return lax.fori_loop(0, 17, pstep, jnp.int32(0))

    p = lax.cond(total_eq == need, lambda: jnp.int32(NROWS), tie_search)
    keep = (u > T) | (eq & (idx <= p))
    nm_ref[...] = keep.astype(jnp.float32)
    sel_ref[...] = jnp.where(keep, s, 0.0)


def _tcB(ms, k):
    return pl.pallas_call(
        functools.partial(_tcB_body, k),
        out_shape=[
            jax.ShapeDtypeStruct((1, NROWS), jnp.float32),
            jax.ShapeDtypeStruct((1, NROWS), jnp.float32),
        ],
    )(ms)


def _tcC_body(xpreT_ref, sel_ref, nm_ref, xa_ref, rmax_ref, rsum_ref):
    i = pl.program_id(0)
    xoutT = xpreT_ref[...] * sel_ref[...]
    nm = nm_ref[...]
    xa_ref[...] = jnp.where(nm > 0, xoutT, NEG).T
    bmax = jnp.max(jnp.where(nm > 0, xoutT, -jnp.inf), axis=1, keepdims=True)
    bsum = jnp.sum(xoutT, axis=1, keepdims=True)

    @pl.when(i == 0)
    def _():
        rmax_ref[...] = bmax
        rsum_ref[...] = bsum

    @pl.when(i > 0)
    def _():
        rmax_ref[...] = jnp.maximum(rmax_ref[...], bmax)
        rsum_ref[...] = rsum_ref[...] + bsum


def _tcC(xpreT, sel, nm):
    return pl.pallas_call(
        _tcC_body,
        grid=(GRID_N,),
        in_specs=[
            pl.BlockSpec((F, BN), lambda i: (0, i)),
            pl.BlockSpec((1, BN), lambda i: (0, i)),
            pl.BlockSpec((1, BN), lambda i: (0, i)),
        ],
        out_specs=[
            pl.BlockSpec((BN, F), lambda i: (i, 0)),
            pl.BlockSpec((F, 1), lambda i: (0, 0)),
            pl.BlockSpec((F, 1), lambda i: (0, 0)),
        ],
        out_shape=[
            jax.ShapeDtypeStruct((NROWS, F), jnp.float32),
            jax.ShapeDtypeStruct((F, 1), jnp.float32),
            jax.ShapeDtypeStruct((F, 1), jnp.float32),
        ],
    )(xpreT, sel, nm)


def _tcD_body(rm0, rs0, rm1, rs1, rm2, rs2, rm3, rs3,
              w1_ref, b1_ref, w2_ref, b2_ref, w3_ref, b3_ref, out_ref):
    reads = None
    for rm, rs, k in ((rm0, rs0, KS[0]), (rm1, rs1, KS[1]),
                      (rm2, rs2, KS[2]), (rm3, rs3, KS[3])):
        r = jnp.concatenate([rm[...].T, rs[...].T / jnp.float32(k)], axis=1)
        reads = r if reads is None else reads + r
    h = jnp.maximum(jnp.dot(reads, w1_ref[...],
                            preferred_element_type=jnp.float32) + b1_ref[...], 0.0)
    h = jnp.maximum(jnp.dot(h, w2_ref[...],
                            preferred_element_type=jnp.float32) + b2_ref[...], 0.0)
    z = jnp.dot(h, w3_ref[...], preferred_element_type=jnp.float32) + b3_ref[...]
    zm = jnp.max(z, axis=1, keepdims=True)
    zs = z - zm
    out_ref[...] = zs - jnp.log(jnp.sum(jnp.exp(zs), axis=1, keepdims=True))


def _tcD(rstats, w1, b1, w2, b2, w3, b3):
    args = []
    for rm, rs in rstats:
        args += [rm, rs]
    return pl.pallas_call(
        _tcD_body,
        out_shape=jax.ShapeDtypeStruct((1, 5), jnp.float32),
    )(*args, w1, b1, w2, b2, w3, b3)


def kernel(x, edge_index, edge_attr, batch, params):
    del batch
    f32 = jnp.float32
    src = edge_index[0]
    dst = edge_index[1]
    padi = (N + (jnp.arange(EPAD - E, dtype=jnp.int32) % NPAD)).astype(jnp.int32)
    srcm = jnp.concatenate([src, padi]).reshape(EROWS, 128)
    dstm = jnp.concatenate([dst, padi]).reshape(EROWS, 128)
    attrm = jnp.concatenate([edge_attr[:, 0],
                             jnp.zeros((EPAD - E,), f32)]).reshape(EROWS, 128)

    xcol = jnp.concatenate([x[:, 0], jnp.full((NPAD,), NEG, f32)])[:, None]
    rest = jnp.concatenate([jnp.zeros((N, F - 1), f32),
                            jnp.full((NPAD, F - 1), NEG, f32)], axis=0)
    x_nm = jnp.concatenate([xcol, rest], axis=1)
    mask = jnp.concatenate([jnp.ones((N,), f32),
                            jnp.zeros((NPAD,), f32)])[None, :]

    rstats = []
    for i, k in enumerate(KS):
        we = params['We%d' % i][0]
        be = params['be%d' % i]
        d = we.shape[0]
        wv = jnp.concatenate([we, jnp.zeros((F - d,), f32)]) if d < F else we
        bv = jnp.concatenate([be, jnp.zeros((F - d,), f32)]) if d < F else be
        wb = jnp.stack([wv, bv])
        wa = params['Wa%d' % i]
        if wa.shape[0] < F:
            wa = jnp.concatenate([wa, jnp.zeros((F - wa.shape[0], F), f32)], axis=0)

        agg2 = _sc_edge(x_nm, srcm, dstm, attrm, wb)
        xpreT, ms = _tcA(x_nm, agg2, mask,
                         wa, params['ba%d' % i][None, :], params['g%d' % i][None, :],
                         params['bb%d' % i][None, :], params['Wb%d' % i],
                         params['b2_%d' % i][None, :], params['p%d' % i][:, None])
        sel, nm = _tcB(ms, k)
        x_nm, rmax, rsum = _tcC(xpreT, sel, nm)
        mask = nm
        rstats.append((rmax, rsum))

    return _tcD(rstats, params['Wl1'], params['bl1'][None, :],
                params['Wl2'], params['bl2'][None, :],
                params['Wl3'], params['bl3'][None, :])


# speedup vs baseline: 4.8336x; 1.0048x over previous
import functools

import jax
import jax.numpy as jnp
from jax import lax
from jax.experimental import pallas as pl
from jax.experimental.pallas import tpu as pltpu
from jax.experimental.pallas import tpu_sc as plsc

N = 50000
F = 16
NPAD = 48
NROWS = N + NPAD
E = 1600000
EROWS = 12544
EPAD = EROWS * 128
NWORK = 32
RW = EROWS // NWORK
CH = 56
NCHUNK = RW // CH
PAIRS = CH // 4
SUBROWS = NROWS // 16
NEG = -1e30
BN = 2944
GRID_N = NROWS // BN
KS = (40000, 32000, 25600, 20480)


def _sc_edge_body(xa, srcm, dstm, attrm, wb, out,
                  sbuf, dbuf, abuf, xb0, xb1, xb2, xb3, wbbuf, zbuf, acc,
                  gs0, gs1, gs2, gs3, ss0, ss1, ss2, ss3):
    c = lax.axis_index("c")
    s = lax.axis_index("s")
    wid = s * 2 + c
    xbufs = (xb0, xb1, xb2, xb3)
    gsems = (gs0, gs1, gs2, gs3)
    ssems = (ss0, ss1, ss2, ss3)

    def zrow(r, _):
        zbuf[r] = jnp.zeros((F,), jnp.float32)
        return 0
    lax.fori_loop(0, SUBROWS, zrow, 0)
    pltpu.sync_copy(zbuf, acc.at[pl.ds(s * SUBROWS, SUBROWS)])

    pltpu.sync_copy(wb, wbbuf)
    plsc.subcore_barrier()

    wv = wbbuf[0]
    bv = wbbuf[1]
    row0 = wid * RW

    def chunk(ci, _):
        r0 = row0 + ci * CH
        pltpu.sync_copy(srcm.at[pl.ds(r0, CH)], sbuf)
        pltpu.sync_copy(dstm.at[pl.ds(r0, CH)], dbuf)
        pltpu.sync_copy(attrm.at[pl.ds(r0, CH)], abuf)

        for u in range(4):
            pltpu.async_copy(xa.at[sbuf.at[u]], xbufs[u], gsems[u])

        def body(t, _):
            for u in range(4):
                j = t * 4 + u
                pltpu.make_async_copy(xa.at[sbuf.at[j]], xbufs[u],
                                      gsems[u]).wait()
                xbuf = xbufs[u]
                for e0 in range(0, 128, 16):
                    avec = abuf[j, pl.ds(e0, 16)]
                    for tt in range(16):
                        e = e0 + tt
                        xbuf[e] = jnp.maximum(
                            xbuf[e] + (avec[tt] * wv + bv), 0.0)
                pltpu.async_copy(xbuf, acc.at[dbuf.at[j]], ssems[u],
                                 add=True)

            @pl.when(t < PAIRS - 1)
            def _():
                for u in range(4):
                    j = t * 4 + u
                    pltpu.make_async_copy(xbufs[u], acc.at[dbuf.at[j]],
                                          ssems[u]).wait()
                    pltpu.async_copy(xa.at[sbuf.at[j + 4]], xbufs[u],
                                     gsems[u])
            return 0
        lax.fori_loop(0, PAIRS, body, 0)

        for u in range(4):
            j = (PAIRS - 1) * 4 + u
            pltpu.make_async_copy(xbufs[u], acc.at[dbuf.at[j]],
                                  ssems[u]).wait()
        return 0
    lax.fori_loop(0, NCHUNK, chunk, 0)

    plsc.subcore_barrier()

    off = s * SUBROWS
    pltpu.sync_copy(acc.at[pl.ds(off, SUBROWS)], zbuf)
    pltpu.sync_copy(zbuf, out.at[c, pl.ds(off, SUBROWS)])


def _sc_edge(xa, srcm, dstm, attrm, wb):
    mesh = plsc.VectorSubcoreMesh(core_axis_name="c", subcore_axis_name="s")
    return pl.kernel(
        _sc_edge_body,
        out_type=jax.ShapeDtypeStruct((2, NROWS, F), jnp.float32),
        mesh=mesh,
        compiler_params=pltpu.CompilerParams(use_tc_tiling_on_sc=False),
        scratch_types=[
            pltpu.VMEM((CH, 128), jnp.int32),
            pltpu.VMEM((CH, 128), jnp.int32),
            pltpu.VMEM((CH, 128), jnp.float32),
            pltpu.VMEM((128, F), jnp.float32),
            pltpu.VMEM((128, F), jnp.float32),
            pltpu.VMEM((128, F), jnp.float32),
            pltpu.VMEM((128, F), jnp.float32),
            pltpu.VMEM((2, F), jnp.float32),
            pltpu.VMEM((SUBROWS, F), jnp.float32),
            pltpu.VMEM_SHARED((NROWS, F), jnp.float32),
        ] + [pltpu.SemaphoreType.DMA] * 8,
    )(xa, srcm, dstm, attrm, wb)


def _tcA_body(x_ref, agg_ref, mask_ref, wa_ref, ba_ref, g_ref, bb_ref,
              wbm_ref, b2_ref, p_ref, xpreT_ref, ms_ref):
    xb = x_ref[...] + agg_ref[0] + agg_ref[1]
    h = jnp.dot(xb, wa_ref[...], preferred_element_type=jnp.float32)
    h = (h + ba_ref[...]) / jnp.sqrt(jnp.float32(1.0 + 1e-5)) * g_ref[...] + bb_ref[...]
    h = jnp.maximum(h, 0.0)
    h = jnp.dot(h, wbm_ref[...], preferred_element_type=jnp.float32) + b2_ref[...]
    xpre = jnp.maximum(h, 0.0)
    xpreT = xpre.T
    xpreT_ref[...] = xpreT
    p = p_ref[...]
    pn = p / jnp.sqrt(jnp.sum(p * p))
    score = jnp.tanh(jnp.sum(xpreT * pn, axis=0, keepdims=True))
    ms_ref[...] = jnp.where(mask_ref[...] > 0, score, -jnp.inf)


def _tcA(x_nm, agg2, mask, wa, ba, g, bb, wbm, b2, p):
    return pl.pallas_call(
        _tcA_body,
        grid=(GRID_N,),
        in_specs=[
            pl.BlockSpec((BN, F), lambda i: (i, 0)),
            pl.BlockSpec((2, BN, F), lambda i: (0, i, 0)),
            pl.BlockSpec((1, BN), lambda i: (0, i)),
            pl.BlockSpec((F, F), lambda i: (0, 0)),
            pl.BlockSpec((1, F), lambda i: (0, 0)),
            pl.BlockSpec((1, F), lambda i: (0, 0)),
            pl.BlockSpec((1, F), lambda i: (0, 0)),
            pl.BlockSpec((F, F), lambda i: (0, 0)),
            pl.BlockSpec((1, F), lambda i: (0, 0)),
            pl.BlockSpec((F, 1), lambda i: (0, 0)),
        ],
        out_specs=[
            pl.BlockSpec((F, BN), lambda i: (0, i)),
            pl.BlockSpec((1, BN), lambda i: (0, i)),
        ],
        out_shape=[
            jax.ShapeDtypeStruct((F, NROWS), jnp.float32),
            jax.ShapeDtypeStruct((1, NROWS), jnp.float32),
        ],
    )(x_nm, agg2, mask, wa, ba, g, bb, wbm, b2, p)


def _pool_search(k, s):
    ub = lax.bitcast_convert_type(s, jnp.uint32)
    sign = ub >> jnp.uint32(31)
    u = jnp.where(sign == jnp.uint32(1), ~ub, ub | jnp.uint32(0x80000000))
    one = jnp.uint32(1)

    def tstep(t, T):
        hi = jnp.uint32(31) - 2 * t.astype(jnp.uint32)
        c1 = T | (one << hi)
        c2 = T | (one << (hi - 1))
        c3 = c1 | (one << (hi - 1))
        n1 = jnp.sum((u >= c1).astype(jnp.int32))
        n2 = jnp.sum((u >= c2).astype(jnp.int32))
        n3 = jnp.sum((u >= c3).astype(jnp.int32))
        return jnp.where(n3 >= k, c3,
                         jnp.where(n1 >= k, c1,
                                   jnp.where(n2 >= k, c2, T)))
    T = lax.fori_loop(0, 16, tstep, jnp.uint32(0))

    cnt_gt = jnp.sum((u > T).astype(jnp.int32))
    total_eq = jnp.sum((u == T).astype(jnp.int32))
    need = k - cnt_gt
    eq = (u == T)
    idx = lax.broadcasted_iota(jnp.int32, (1, NROWS), 1)

    def tie_search():
        def pstep(t, p):
            trial = p + (jnp.int32(1) << (jnp.int32(16) - t))
            g = jnp.sum((eq & (idx < trial)).astype(jnp.int32))
            return jnp.where(g < need, trial, p)
        return lax.fori_loop(0, 17, pstep, jnp.int32(0))

    p = lax.cond(total_eq == need, lambda: jnp.int32(NROWS), tie_search)
    return (u > T) | (eq & (idx <= p))


def _tcC_body(k, xpreT_ref, ms_ref, xa_ref, nm_ref, rmax_ref, rsum_ref,
              sel_scr, nm_scr):
    i = pl.program_id(0)

    @pl.when(i == 0)
    def _():
        s = ms_ref[...]
        keep = _pool_search(k, s)
        nmf = keep.astype(jnp.float32)
        sel_scr[...] = jnp.where(keep, s, 0.0)
        nm_scr[...] = nmf
        nm_ref[...] = nmf

    sel = sel_scr[:, pl.ds(i * BN, BN)]
    nm = nm_scr[:, pl.ds(i * BN, BN)]
    xoutT = xpreT_ref[...] * sel
    xa_ref[...] = jnp.where(nm > 0, xoutT, NEG).T
    bmax = jnp.max(jnp.where(nm > 0, xoutT, -jnp.inf), axis=1, keepdims=True)
    bsum = jnp.sum(xoutT, axis=1, keepdims=True)

    @pl.when(i == 0)
    def _():
        rmax_ref[...] = bmax
        rsum_ref[...] = bsum

    @pl.when(i > 0)
    def _():
        rmax_ref[...] = jnp.maximum(rmax_ref[...], bmax)
        rsum_ref[...] = rsum_ref[...] + bsum


def _tcC(xpreT, ms, k):
    return pl.pallas_call(
        functools.partial(_tcC_body, k),
        grid=(GRID_N,),
        in_specs=[
            pl.BlockSpec((F, BN), lambda i: (0, i)),
            pl.BlockSpec((1, NROWS), lambda i: (0, 0)),
        ],
        out_specs=[
            pl.BlockSpec((BN, F), lambda i: (i, 0)),
            pl.BlockSpec((1, NROWS), lambda i: (0, 0)),
            pl.BlockSpec((F, 1), lambda i: (0, 0)),
            pl.BlockSpec((F, 1), lambda i: (0, 0)),
        ],
        out_shape=[
            jax.ShapeDtypeStruct((NROWS, F), jnp.float32),
            jax.ShapeDtypeStruct((1, NROWS), jnp.float32),
            jax.ShapeDtypeStruct((F, 1), jnp.float32),
            jax.ShapeDtypeStruct((F, 1), jnp.float32),
        ],
        scratch_shapes=[
            pltpu.VMEM((1, NROWS), jnp.float32),
            pltpu.VMEM((1, NROWS), jnp.float32),
        ],
    )(xpreT, ms)


def _tcD_body(rm0, rs0, rm1, rs1, rm2, rs2, rm3, rs3,
              w1_ref, b1_ref, w2_ref, b2_ref, w3_ref, b3_ref, out_ref):
    reads = None
    for rm, rs, k in ((rm0, rs0, KS[0]), (rm1, rs1, KS[1]),
                      (rm2, rs2, KS[2]), (rm3, rs3, KS[3])):
        r = jnp.concatenate([rm[...].T, rs[...].T / jnp.float32(k)], axis=1)
        reads = r if reads is None else reads + r
    h = jnp.maximum(jnp.dot(reads, w1_ref[...],
                            preferred_element_type=jnp.float32) + b1_ref[...], 0.0)
    h = jnp.maximum(jnp.dot(h, w2_ref[...],
                            preferred_element_type=jnp.float32) + b2_ref[...], 0.0)
    z = jnp.dot(h, w3_ref[...], preferred_element_type=jnp.float32) + b3_ref[...]
    zm = jnp.max(z, axis=1, keepdims=True)
    zs = z - zm
    out_ref[...] = zs - jnp.log(jnp.sum(jnp.exp(zs), axis=1, keepdims=True))


def _tcD(rstats, w1, b1, w2, b2, w3, b3):
    args = []
    for rm, rs in rstats:
        args += [rm, rs]
    return pl.pallas_call(
        _tcD_body,
        out_shape=jax.ShapeDtypeStruct((1, 5), jnp.float32),
    )(*args, w1, b1, w2, b2, w3, b3)


def kernel(x, edge_index, edge_attr, batch, params):
    del batch
    f32 = jnp.float32
    src = edge_index[0]
    dst = edge_index[1]
    padi = (N + (jnp.arange(EPAD - E, dtype=jnp.int32) % NPAD)).astype(jnp.int32)
    srcm = jnp.concatenate([src, padi]).reshape(EROWS, 128)
    dstm = jnp.concatenate([dst, padi]).reshape(EROWS, 128)
    attrm = jnp.concatenate([edge_attr[:, 0],
                             jnp.zeros((EPAD - E,), f32)]).reshape(EROWS, 128)

    xcol = jnp.concatenate([x[:, 0], jnp.full((NPAD,), NEG, f32)])[:, None]
    rest = jnp.concatenate([jnp.zeros((N, F - 1), f32),
                            jnp.full((NPAD, F - 1), NEG, f32)], axis=0)
    x_nm = jnp.concatenate([xcol, rest], axis=1)
    mask = jnp.concatenate([jnp.ones((N,), f32),
                            jnp.zeros((NPAD,), f32)])[None, :]

    rstats = []
    for i, k in enumerate(KS):
        we = params['We%d' % i][0]
        be = params['be%d' % i]
        d = we.shape[0]
        wv = jnp.concatenate([we, jnp.zeros((F - d,), f32)]) if d < F else we
        bv = jnp.concatenate([be, jnp.zeros((F - d,), f32)]) if d < F else be
        wb = jnp.stack([wv, bv])
        wa = params['Wa%d' % i]
        if wa.shape[0] < F:
            wa = jnp.concatenate([wa, jnp.zeros((F - wa.shape[0], F), f32)], axis=0)

        agg2 = _sc_edge(x_nm, srcm, dstm, attrm, wb)
        xpreT, ms = _tcA(x_nm, agg2, mask,
                         wa, params['ba%d' % i][None, :], params['g%d' % i][None, :],
                         params['bb%d' % i][None, :], params['Wb%d' % i],
                         params['b2_%d' % i][None, :], params['p%d' % i][:, None])
        x_nm, nm, rmax, rsum = _tcC(xpreT, ms, k)
        mask = nm
        rstats.append((rmax, rsum))

    return _tcD(rstats, params['Wl1'], params['bl1'][None, :],
                params['Wl2'], params['bl2'][None, :],
                params['Wl3'], params['bl3'][None, :])


# speedup vs baseline: 5.4835x; 1.1344x over previous
import functools

import jax
import jax.numpy as jnp
from jax import lax
from jax.experimental import pallas as pl
from jax.experimental.pallas import tpu as pltpu
from jax.experimental.pallas import tpu_sc as plsc

N = 50000
F = 16
NPAD = 48
NROWS = N + NPAD
E = 1600000
EROWS = 12544
EPAD = EROWS * 128
NWORK = 32
RW = EROWS // NWORK
CH = 56
NCHUNK = RW // CH
PAIRS = CH // 8
SUBROWS = NROWS // 16
NEG = -1e30
BN = 2944
GRID_N = NROWS // BN
KS = (40000, 32000, 25600, 20480)


def _sc_edge_body(xa, srcm, dstm, attrm, wb, out,
                  sbuf, dbuf, abuf, xb0, xb1, xb2, xb3, xb4, xb5, xb6, xb7,
                  wbbuf, zbuf, acc,
                  gs0, gs1, gs2, gs3, gs4, gs5, gs6, gs7,
                  ss0, ss1, ss2, ss3, ss4, ss5, ss6, ss7):
    c = lax.axis_index("c")
    s = lax.axis_index("s")
    wid = s * 2 + c
    xbufs = (xb0, xb1, xb2, xb3, xb4, xb5, xb6, xb7)
    gsems = (gs0, gs1, gs2, gs3, gs4, gs5, gs6, gs7)
    ssems = (ss0, ss1, ss2, ss3, ss4, ss5, ss6, ss7)

    HS = SUBROWS // 2
    def zrow(r, _):
        zbuf[r] = jnp.zeros((F,), jnp.float32)
        return 0
    lax.fori_loop(0, HS, zrow, 0)
    pltpu.sync_copy(zbuf, acc.at[pl.ds(s * SUBROWS, HS)])
    pltpu.sync_copy(zbuf, acc.at[pl.ds(s * SUBROWS + HS, HS)])

    pltpu.sync_copy(wb, wbbuf)
    plsc.subcore_barrier()

    wv = wbbuf[0]
    bv = wbbuf[1]
    row0 = wid * RW

    def chunk(ci, _):
        r0 = row0 + ci * CH
        pltpu.sync_copy(srcm.at[pl.ds(r0, CH)], sbuf)
        pltpu.sync_copy(dstm.at[pl.ds(r0, CH)], dbuf)
        pltpu.sync_copy(attrm.at[pl.ds(r0, CH)], abuf)

        for u in range(8):
            pltpu.async_copy(xa.at[sbuf.at[u]], xbufs[u], gsems[u])

        def body(t, _):
            for u in range(8):
                j = t * 8 + u
                pltpu.make_async_copy(xa.at[sbuf.at[j]], xbufs[u],
                                      gsems[u]).wait()
                xbuf = xbufs[u]

                def grp(g, _):
                    avec = abuf[j, pl.ds(g * 16, 16)]
                    for tt in range(16):
                        xbuf[pl.ds(g * 16 + tt, 1)] = jnp.maximum(
                            xbuf[pl.ds(g * 16 + tt, 1)]
                            + (avec[tt] * wv + bv), 0.0)
                    return 0
                lax.fori_loop(0, 8, grp, 0)
                pltpu.async_copy(xbuf, acc.at[dbuf.at[j]], ssems[u],
                                 add=True)

            @pl.when(t < PAIRS - 1)
            def _():
                for u in range(8):
                    j = t * 8 + u
                    pltpu.make_async_copy(xbufs[u], acc.at[dbuf.at[j]],
                                          ssems[u]).wait()
                    pltpu.async_copy(xa.at[sbuf.at[j + 8]], xbufs[u],
                                     gsems[u])
            return 0
        lax.fori_loop(0, PAIRS, body, 0)

        for u in range(8):
            j = (PAIRS - 1) * 8 + u
            pltpu.make_async_copy(xbufs[u], acc.at[dbuf.at[j]],
                                  ssems[u]).wait()
        return 0
    lax.fori_loop(0, NCHUNK, chunk, 0)

    plsc.subcore_barrier()

    for h in range(2):
        off = s * SUBROWS + h * HS
        pltpu.sync_copy(acc.at[pl.ds(off, HS)], zbuf)
        pltpu.sync_copy(zbuf, out.at[c, pl.ds(off, HS)])


def _sc_edge(xa, srcm, dstm, attrm, wb):
    mesh = plsc.VectorSubcoreMesh(core_axis_name="c", subcore_axis_name="s")
    return pl.kernel(
        _sc_edge_body,
        out_type=jax.ShapeDtypeStruct((2, NROWS, F), jnp.float32),
        mesh=mesh,
        compiler_params=pltpu.CompilerParams(use_tc_tiling_on_sc=False),
        scratch_types=[
            pltpu.VMEM((CH, 128), jnp.int32),
            pltpu.VMEM((CH, 128), jnp.int32),
            pltpu.VMEM((CH, 128), jnp.float32),
            pltpu.VMEM((128, F), jnp.float32),
            pltpu.VMEM((128, F), jnp.float32),
            pltpu.VMEM((128, F), jnp.float32),
            pltpu.VMEM((128, F), jnp.float32),
            pltpu.VMEM((128, F), jnp.float32),
            pltpu.VMEM((128, F), jnp.float32),
            pltpu.VMEM((128, F), jnp.float32),
            pltpu.VMEM((128, F), jnp.float32),
            pltpu.VMEM((2, F), jnp.float32),
            pltpu.VMEM((SUBROWS // 2, F), jnp.float32),
            pltpu.VMEM_SHARED((NROWS, F), jnp.float32),
        ] + [pltpu.SemaphoreType.DMA] * 16,
    )(xa, srcm, dstm, attrm, wb)


def _tcA_body(x_ref, agg_ref, mask_ref, wa_ref, ba_ref, g_ref, bb_ref,
              wbm_ref, b2_ref, p_ref, xpreT_ref, ms_ref):
    xb = x_ref[...] + agg_ref[0] + agg_ref[1]
    h = jnp.dot(xb, wa_ref[...], preferred_element_type=jnp.float32)
    h = (h + ba_ref[...]) / jnp.sqrt(jnp.float32(1.0 + 1e-5)) * g_ref[...] + bb_ref[...]
    h = jnp.maximum(h, 0.0)
    h = jnp.dot(h, wbm_ref[...], preferred_element_type=jnp.float32) + b2_ref[...]
    xpre = jnp.maximum(h, 0.0)
    xpreT = xpre.T
    xpreT_ref[...] = xpreT
    p = p_ref[...]
    pn = p / jnp.sqrt(jnp.sum(p * p))
    score = jnp.tanh(jnp.sum(xpreT * pn, axis=0, keepdims=True))
    ms_ref[...] = jnp.where(mask_ref[...] > 0, score, -jnp.inf)


def _tcA(x_nm, agg2, mask, wa, ba, g, bb, wbm, b2, p):
    return pl.pallas_call(
        _tcA_body,
        grid=(GRID_N,),
        in_specs=[
            pl.BlockSpec((BN, F), lambda i: (i, 0)),
            pl.BlockSpec((2, BN, F), lambda i: (0, i, 0)),
            pl.BlockSpec((1, BN), lambda i: (0, i)),
            pl.BlockSpec((F, F), lambda i: (0, 0)),
            pl.BlockSpec((1, F), lambda i: (0, 0)),
            pl.BlockSpec((1, F), lambda i: (0, 0)),
            pl.BlockSpec((1, F), lambda i: (0, 0)),
            pl.BlockSpec((F, F), lambda i: (0, 0)),
            pl.BlockSpec((1, F), lambda i: (0, 0)),
            pl.BlockSpec((F, 1), lambda i: (0, 0)),
        ],
        out_specs=[
            pl.BlockSpec((F, BN), lambda i: (0, i)),
            pl.BlockSpec((1, BN), lambda i: (0, i)),
        ],
        out_shape=[
            jax.ShapeDtypeStruct((F, NROWS), jnp.float32),
            jax.ShapeDtypeStruct((1, NROWS), jnp.float32),
        ],
    )(x_nm, agg2, mask, wa, ba, g, bb, wbm, b2, p)


def _pool_search(k, s):
    ub = lax.bitcast_convert_type(s, jnp.uint32)
    sign = ub >> jnp.uint32(31)
    u = jnp.where(sign == jnp.uint32(1), ~ub, ub | jnp.uint32(0x80000000))
    one = jnp.uint32(1)

    def tstep(t, T):
        hi = jnp.uint32(31) - 2 * t.astype(jnp.uint32)
        c1 = T | (one << hi)
        c2 = T | (one << (hi - 1))
        c3 = c1 | (one << (hi - 1))
        n1 = jnp.sum((u >= c1).astype(jnp.int32))
        n2 = jnp.sum((u >= c2).astype(jnp.int32))
        n3 = jnp.sum((u >= c3).astype(jnp.int32))
        return jnp.where(n3 >= k, c3,
                         jnp.where(n1 >= k, c1,
                                   jnp.where(n2 >= k, c2, T)))
    T = lax.fori_loop(0, 16, tstep, jnp.uint32(0))

    cnt_gt = jnp.sum((u > T).astype(jnp.int32))
    total_eq = jnp.sum((u == T).astype(jnp.int32))
    need = k - cnt_gt
    eq = (u == T)
    idx = lax.broadcasted_iota(jnp.int32, (1, NROWS), 1)

    def tie_search():
        def pstep(t, p):
            trial = p + (jnp.int32(1) << (jnp.int32(16) - t))
            g = jnp.sum((eq & (idx < trial)).astype(jnp.int32))
            return jnp.where(g < need, trial, p)
        return lax.fori_loop(0, 17, pstep, jnp.int32(0))

    p = lax.cond(total_eq == need, lambda: jnp.int32(NROWS), tie_search)
    return (u > T) | (eq & (idx <= p))


def _tcC_body(k, xpreT_ref, ms_ref, xa_ref, nm_ref, rmax_ref, rsum_ref,
              sel_scr, nm_scr):
    i = pl.program_id(0)

    @pl.when(i == 0)
    def _():
        s = ms_ref[...]
        keep = _pool_search(k, s)
        nmf = keep.astype(jnp.float32)
        sel_scr[...] = jnp.where(keep, s, 0.0)
        nm_scr[...] = nmf
        nm_ref[...] = nmf

    sel = sel_scr[:, pl.ds(i * BN, BN)]
    nm = nm_scr[:, pl.ds(i * BN, BN)]
    xoutT = xpreT_ref[...] * sel
    xa_ref[...] = jnp.where(nm > 0, xoutT, NEG).T
    bmax = jnp.max(jnp.where(nm > 0, xoutT, -jnp.inf), axis=1, keepdims=True)
    bsum = jnp.sum(xoutT, axis=1, keepdims=True)

    @pl.when(i == 0)
    def _():
        rmax_ref[...] = bmax
        rsum_ref[...] = bsum

    @pl.when(i > 0)
    def _():
        rmax_ref[...] = jnp.maximum(rmax_ref[...], bmax)
        rsum_ref[...] = rsum_ref[...] + bsum


def _tcC(xpreT, ms, k):
    return pl.pallas_call(
        functools.partial(_tcC_body, k),
        grid=(GRID_N,),
        in_specs=[
            pl.BlockSpec((F, BN), lambda i: (0, i)),
            pl.BlockSpec((1, NROWS), lambda i: (0, 0)),
        ],
        out_specs=[
            pl.BlockSpec((BN, F), lambda i: (i, 0)),
            pl.BlockSpec((1, NROWS), lambda i: (0, 0)),
            pl.BlockSpec((F, 1), lambda i: (0, 0)),
            pl.BlockSpec((F, 1), lambda i: (0, 0)),
        ],
        out_shape=[
            jax.ShapeDtypeStruct((NROWS, F), jnp.float32),
            jax.ShapeDtypeStruct((1, NROWS), jnp.float32),
            jax.ShapeDtypeStruct((F, 1), jnp.float32),
            jax.ShapeDtypeStruct((F, 1), jnp.float32),
        ],
        scratch_shapes=[
            pltpu.VMEM((1, NROWS), jnp.float32),
            pltpu.VMEM((1, NROWS), jnp.float32),
        ],
    )(xpreT, ms)


def _tcD_body(rm0, rs0, rm1, rs1, rm2, rs2, rm3, rs3,
              w1_ref, b1_ref, w2_ref, b2_ref, w3_ref, b3_ref, out_ref):
    reads = None
    for rm, rs, k in ((rm0, rs0, KS[0]), (rm1, rs1, KS[1]),
                      (rm2, rs2, KS[2]), (rm3, rs3, KS[3])):
        r = jnp.concatenate([rm[...].T, rs[...].T / jnp.float32(k)], axis=1)
        reads = r if reads is None else reads + r
    h = jnp.maximum(jnp.dot(reads, w1_ref[...],
                            preferred_element_type=jnp.float32) + b1_ref[...], 0.0)
    h = jnp.maximum(jnp.dot(h, w2_ref[...],
                            preferred_element_type=jnp.float32) + b2_ref[...], 0.0)
    z = jnp.dot(h, w3_ref[...], preferred_element_type=jnp.float32) + b3_ref[...]
    zm = jnp.max(z, axis=1, keepdims=True)
    zs = z - zm
    out_ref[...] = zs - jnp.log(jnp.sum(jnp.exp(zs), axis=1, keepdims=True))


def _tcD(rstats, w1, b1, w2, b2, w3, b3):
    args = []
    for rm, rs in rstats:
        args += [rm, rs]
    return pl.pallas_call(
        _tcD_body,
        out_shape=jax.ShapeDtypeStruct((1, 5), jnp.float32),
    )(*args, w1, b1, w2, b2, w3, b3)


def kernel(x, edge_index, edge_attr, batch, params):
    del batch
    f32 = jnp.float32
    src = edge_index[0]
    dst = edge_index[1]
    padi = (N + (jnp.arange(EPAD - E, dtype=jnp.int32) % NPAD)).astype(jnp.int32)
    srcm = jnp.concatenate([src, padi]).reshape(EROWS, 128)
    dstm = jnp.concatenate([dst, padi]).reshape(EROWS, 128)
    attrm = jnp.concatenate([edge_attr[:, 0],
                             jnp.zeros((EPAD - E,), f32)]).reshape(EROWS, 128)

    xcol = jnp.concatenate([x[:, 0], jnp.full((NPAD,), NEG, f32)])[:, None]
    rest = jnp.concatenate([jnp.zeros((N, F - 1), f32),
                            jnp.full((NPAD, F - 1), NEG, f32)], axis=0)
    x_nm = jnp.concatenate([xcol, rest], axis=1)
    mask = jnp.concatenate([jnp.ones((N,), f32),
                            jnp.zeros((NPAD,), f32)])[None, :]

    rstats = []
    for i, k in enumerate(KS):
        we = params['We%d' % i][0]
        be = params['be%d' % i]
        d = we.shape[0]
        wv = jnp.concatenate([we, jnp.zeros((F - d,), f32)]) if d < F else we
        bv = jnp.concatenate([be, jnp.zeros((F - d,), f32)]) if d < F else be
        wb = jnp.stack([wv, bv])
        wa = params['Wa%d' % i]
        if wa.shape[0] < F:
            wa = jnp.concatenate([wa, jnp.zeros((F - wa.shape[0], F), f32)], axis=0)

        agg2 = _sc_edge(x_nm, srcm, dstm, attrm, wb)
        xpreT, ms = _tcA(x_nm, agg2, mask,
                         wa, params['ba%d' % i][None, :], params['g%d' % i][None, :],
                         params['bb%d' % i][None, :], params['Wb%d' % i],
                         params['b2_%d' % i][None, :], params['p%d' % i][:, None])
        x_nm, nm, rmax, rsum = _tcC(xpreT, ms, k)
        mask = nm
        rstats.append((rmax, rsum))

    return _tcD(rstats, params['Wl1'], params['bl1'][None, :],
                params['Wl2'], params['bl2'][None, :],
                params['Wl3'], params['bl3'][None, :])
